# baseline (speedup 1.0000x reference)
import jax
import jax.numpy as jnp
import numpy as np
from jax import lax
from jax.experimental import pallas as pl
from jax.experimental.pallas import tpu as pltpu

DIM = 1024
SD = 128
SHID = 256
NE = 8
B = 4
S = 2048
TCHUNK = 128
NT = S // TCHUNK
TOK = B * TCHUNK


_i0 = np.int32(0)


def _routes(tok_u32):
    x = tok_u32
    x = x ^ (x >> 16)
    x = x * jnp.uint32(2246822507)
    x = x ^ (x >> 13)
    x = x * jnp.uint32(3266489909)
    x = x ^ (x >> 16)
    return (x & jnp.uint32(7)).astype(jnp.int32)


def _dot_nt(a, b):
    return lax.dot_general(a, b, (((1,), (1,)), ((), ())),
                           preferred_element_type=jnp.float32)


def _shift_down(x, s, fill):
    rolled = pltpu.roll(x, jnp.int32(s), 0)
    tmod = lax.rem(lax.broadcasted_iota(jnp.int32, x.shape, 0),
                   jnp.int32(TCHUNK))
    return jnp.where(tmod >= s, rolled, fill)


def _ssm_body(tok_ref, x_ref, wi_ref, wsi_ref, wso_ref, wo_ref, d_ref,
              out_ref, h_ref):
    t = pl.program_id(0)

    @pl.when(t == 0)
    def _():
        h_ref[...] = jnp.zeros_like(h_ref)

    r = _routes(tok_ref[0])
    xc = x_ref[...].reshape(TOK, DIM)

    sel = jnp.zeros((TOK, 4 * SD), jnp.float32)
    u = jnp.zeros((TOK, SD), jnp.float32)
    for e in range(NE):
        m = r == e
        u_e = _dot_nt(xc, wi_ref[e])
        sh_e = _dot_nt(xc, wsi_ref[e])
        sh_e = sh_e * jax.nn.sigmoid(sh_e)
        sel_e = _dot_nt(sh_e, wso_ref[e])
        sel = sel + jnp.where(m, sel_e, 0.0)
        u = u + jnp.where(m, u_e, 0.0)

    a = jax.nn.sigmoid(sel[:, :SD])
    b = jnp.tanh(sel[:, SD:2 * SD])
    c = jnp.tanh(sel[:, 2 * SD:3 * SD])
    dd = jax.nn.sigmoid(sel[:, 3 * SD:])
    v = b * u

    lane_e = lax.broadcasted_iota(jnp.int32, (TOK, NE * SD), 1) // SD
    eq = lane_e == r
    a_rep = jnp.concatenate([a] * NE, axis=1)
    v_rep = jnp.concatenate([v] * NE, axis=1)
    A = jnp.where(eq, a_rep, 1.0)
    V = jnp.where(eq, v_rep, 0.0)

    s = 1
    while s < TCHUNK:
        Ash = _shift_down(A, s, 1.0)
        Vsh = _shift_down(V, s, 0.0)
        V = V + A * Vsh
        A = A * Ash
        s *= 2

    h_enter = jnp.broadcast_to(h_ref[...][:, None, :],
                               (B, TCHUNK, NE * SD)).reshape(TOK, NE * SD)
    h_all = V + A * h_enter
    h_ref[...] = h_all.reshape(B, TCHUNK, NE * SD)[:, TCHUNK - 1]

    h_sel = jnp.zeros((TOK, SD), jnp.float32)
    d_sel = jnp.zeros((TOK, SD), jnp.float32)
    for e in range(NE):
        m = r == e
        h_sel = h_sel + jnp.where(m, h_all[:, e * SD:(e + 1) * SD], 0.0)
        d_sel = d_sel + jnp.where(m, d_ref[e][None, :], 0.0)

    y = c * h_sel + d_sel * dd * u

    out_acc = jnp.zeros((TOK, DIM), jnp.float32)
    for e in range(NE):
        ym = jnp.where(r == e, y, 0.0)
        out_acc = out_acc + _dot_nt(ym, wo_ref[e])
    out_ref[...] = out_acc.reshape(B, TCHUNK, DIM)


def _to_f32(w):
    return w.astype(jnp.float32)
    if False:
        pass
    hi = lax.bitcast_convert_type(w, jnp.uint32)[..., 1]
    s = hi & jnp.uint32(0x80000000)
    e11 = (hi >> 20) & jnp.uint32(0x7FF)
    m20 = hi & jnp.uint32(0xFFFFF)
    bits = s | ((e11 - jnp.uint32(896)) << 23) | (m20 << 3)
    return lax.bitcast_convert_type(jnp.where(e11 <= 896, s, bits),
                                    jnp.float32)


def _to_out_dtype(y, out_dtype):
    if out_dtype != jnp.float64:
        return y.astype(out_dtype)
    b = lax.bitcast_convert_type(y, jnp.uint32)
    s = b & jnp.uint32(0x80000000)
    e8 = (b >> 23) & jnp.uint32(0xFF)
    m = b & jnp.uint32(0x7FFFFF)
    hi = s | ((e8 + jnp.uint32(896)) << 20) | (m >> 3)
    lo = m << 29
    hi = jnp.where(e8 == 0, s, hi)
    lo = jnp.where(e8 == 0, jnp.uint32(0), lo)
    return lax.bitcast_convert_type(jnp.stack([lo, hi], axis=-1),
                                    jnp.float64)


def kernel(x, token_ids, W_in, W_si, W_so, W_out, d_param):
    out_dtype = jnp.result_type(W_out.dtype, x.dtype)
    tok_col = (token_ids.astype(jnp.uint32)
               .reshape(B, NT, TCHUNK).transpose(1, 0, 2)
               .reshape(NT, TOK, 1))
    x = _to_f32(x)
    W_in = _to_f32(W_in)
    W_si = _to_f32(W_si)
    W_so = _to_f32(W_so)
    W_out = _to_f32(W_out)
    d_param = _to_f32(d_param)
    out = pl.pallas_call(
        _ssm_body,
        grid=(NT,),
        in_specs=[
            pl.BlockSpec((1, TOK, 1), lambda t: (t, _i0, _i0)),
            pl.BlockSpec((B, TCHUNK, DIM), lambda t: (_i0, t, _i0)),
            pl.BlockSpec((NE, SD, DIM), lambda t: (_i0, _i0, _i0)),
            pl.BlockSpec((NE, SHID, DIM), lambda t: (_i0, _i0, _i0)),
            pl.BlockSpec((NE, 4 * SD, SHID), lambda t: (_i0, _i0, _i0)),
            pl.BlockSpec((NE, DIM, SD), lambda t: (_i0, _i0, _i0)),
            pl.BlockSpec((NE, SD), lambda t: (_i0, _i0)),
        ],
        out_specs=pl.BlockSpec((B, TCHUNK, DIM), lambda t: (_i0, t, _i0)),
        out_shape=jax.ShapeDtypeStruct((B, S, DIM), jnp.float32),
        scratch_shapes=[pltpu.VMEM((B, NE * SD), jnp.float32)],
        compiler_params=pltpu.CompilerParams(
            dimension_semantics=("arbitrary",)),
    )(tok_col, x, W_in, W_si, W_so, W_out, d_param)
    return _to_out_dtype(out, out_dtype)


# speedup vs baseline: 1.0570x; 1.0570x over previous
import jax
import jax.numpy as jnp
import numpy as np
from jax import lax
from jax.experimental import pallas as pl
from jax.experimental.pallas import tpu as pltpu

DIM = 1024
SD = 128
SHID = 256
NE = 8
B = 4
S = 2048
TCHUNK = 128
NT = S // TCHUNK
TOK = B * TCHUNK


_i0 = np.int32(0)


def _routes(tok_u32):
    x = tok_u32
    x = x ^ (x >> 16)
    x = x * jnp.uint32(2246822507)
    x = x ^ (x >> 13)
    x = x * jnp.uint32(3266489909)
    x = x ^ (x >> 16)
    return (x & jnp.uint32(7)).astype(jnp.int32)


def _dot_nt(a, b):
    return lax.dot_general(a, b, (((1,), (1,)), ((), ())),
                           preferred_element_type=jnp.float32)


def _shift_down(x, s, fill):
    rolled = pltpu.roll(x, jnp.int32(s), 0)
    tmod = lax.rem(lax.broadcasted_iota(jnp.int32, x.shape, 0),
                   jnp.int32(TCHUNK))
    return jnp.where(tmod >= s, rolled, fill)


def _ssm_body(tok_ref, x_ref, wi_ref, wsi_ref, wso_ref, wo_ref, d_ref,
              out_ref, h_ref):
    t = pl.program_id(0)

    @pl.when(t == 0)
    def _():
        h_ref[...] = jnp.zeros_like(h_ref)

    r = _routes(tok_ref[0])
    xc = x_ref[...].reshape(TOK, DIM)

    sel = jnp.zeros((TOK, 4 * SD), jnp.float32)
    u = jnp.zeros((TOK, SD), jnp.float32)
    for e in range(NE):
        m = r == e
        u_e = _dot_nt(xc, wi_ref[e])
        sh_e = _dot_nt(xc, wsi_ref[e])
        sh_e = sh_e * jax.nn.sigmoid(sh_e)
        sel_e = _dot_nt(sh_e, wso_ref[e])
        sel = sel + jnp.where(m, sel_e, 0.0)
        u = u + jnp.where(m, u_e, 0.0)

    a = jax.nn.sigmoid(sel[:, :SD])
    b = jnp.tanh(sel[:, SD:2 * SD])
    c = jnp.tanh(sel[:, 2 * SD:3 * SD])
    dd = jax.nn.sigmoid(sel[:, 3 * SD:])
    v = b * u

    lane_e = lax.broadcasted_iota(jnp.int32, (TOK, NE * SD), 1) // SD
    eq = lane_e == r
    a_rep = jnp.concatenate([a] * NE, axis=1)
    v_rep = jnp.concatenate([v] * NE, axis=1)
    A = jnp.where(eq, a_rep, 1.0)
    V = jnp.where(eq, v_rep, 0.0)

    s = 1
    while s < TCHUNK:
        Ash = _shift_down(A, s, 1.0)
        Vsh = _shift_down(V, s, 0.0)
        V = V + A * Vsh
        A = A * Ash
        s *= 2

    h_enter = jnp.broadcast_to(h_ref[...][:, None, :],
                               (B, TCHUNK, NE * SD)).reshape(TOK, NE * SD)
    h_all = V + A * h_enter
    h_ref[...] = h_all.reshape(B, TCHUNK, NE * SD)[:, TCHUNK - 1]

    h_sel = jnp.zeros((TOK, SD), jnp.float32)
    d_sel = jnp.zeros((TOK, SD), jnp.float32)
    for e in range(NE):
        m = r == e
        h_sel = h_sel + jnp.where(m, h_all[:, e * SD:(e + 1) * SD], 0.0)
        d_sel = d_sel + jnp.where(m, d_ref[e][None, :], 0.0)

    y = c * h_sel + d_sel * dd * u

    out_acc = jnp.zeros((TOK, DIM), jnp.float32)
    for e in range(NE):
        ym = jnp.where(r == e, y, 0.0)
        out_acc = out_acc + _dot_nt(ym, wo_ref[e])

    out_ref[...] = out_acc.reshape(B, TCHUNK, DIM)


def _to_f32(w):
    return w.astype(jnp.float32)
    if False:
        pass
    hi = (lax.bitcast_convert_type(w, jnp.uint64) >> 32).astype(jnp.uint32)
    s = hi & jnp.uint32(0x80000000)
    e11 = (hi >> 20) & jnp.uint32(0x7FF)
    m20 = hi & jnp.uint32(0xFFFFF)
    bits = s | ((e11 - jnp.uint32(896)) << 23) | (m20 << 3)
    return lax.bitcast_convert_type(jnp.where(e11 <= 896, s, bits),
                                    jnp.float32)


def _to_out_dtype(y, out_dtype):
    if out_dtype != jnp.float64:
        return y.astype(out_dtype)
    b = lax.bitcast_convert_type(y, jnp.uint32)
    s = b & jnp.uint32(0x80000000)
    e8 = (b >> 23) & jnp.uint32(0xFF)
    m = b & jnp.uint32(0x7FFFFF)
    hi = s | ((e8 + jnp.uint32(896)) << 20) | (m >> 3)
    lo = m << 29
    hi = jnp.where(e8 == 0, s, hi)
    lo = jnp.where(e8 == 0, jnp.uint32(0), lo)
    w64 = (hi.astype(jnp.uint64) << 32) | lo.astype(jnp.uint64)
    return lax.bitcast_convert_type(w64, jnp.float64)


def kernel(x, token_ids, W_in, W_si, W_so, W_out, d_param):
    out_dtype = jnp.result_type(W_out.dtype, x.dtype)
    tok_col = (token_ids.astype(jnp.uint32)
               .reshape(B, NT, TCHUNK).transpose(1, 0, 2)
               .reshape(NT, TOK, 1))
    x = _to_f32(x)
    W_in = _to_f32(W_in)
    W_si = _to_f32(W_si)
    W_so = _to_f32(W_so)
    W_out = _to_f32(W_out)
    d_param = _to_f32(d_param)
    out = pl.pallas_call(
        _ssm_body,
        grid=(NT,),
        in_specs=[
            pl.BlockSpec((1, TOK, 1), lambda t: (t, _i0, _i0)),
            pl.BlockSpec((B, TCHUNK, DIM), lambda t: (_i0, t, _i0)),
            pl.BlockSpec((NE, SD, DIM), lambda t: (_i0, _i0, _i0)),
            pl.BlockSpec((NE, SHID, DIM), lambda t: (_i0, _i0, _i0)),
            pl.BlockSpec((NE, 4 * SD, SHID), lambda t: (_i0, _i0, _i0)),
            pl.BlockSpec((NE, DIM, SD), lambda t: (_i0, _i0, _i0)),
            pl.BlockSpec((NE, SD), lambda t: (_i0, _i0)),
        ],
        out_specs=pl.BlockSpec((B, TCHUNK, DIM), lambda t: (_i0, t, _i0)),
        out_shape=jax.ShapeDtypeStruct((B, S, DIM), jnp.float32),
        scratch_shapes=[pltpu.VMEM((B, NE * SD), jnp.float32)],
        compiler_params=pltpu.CompilerParams(
            dimension_semantics=("arbitrary",)),
    )(tok_col, x, W_in, W_si, W_so, W_out, d_param)
    return _to_out_dtype(out, out_dtype)


# speedup vs baseline: 1.1513x; 1.0892x over previous
import jax
import jax.numpy as jnp
import numpy as np
from jax import lax
from jax.experimental import pallas as pl
from jax.experimental.pallas import tpu as pltpu
from jax.experimental.pallas import tpu_sc as plsc
import functools

DIM = 1024
SD = 128
SHID = 256
NE = 8
B = 4
S = 2048
TCHUNK = 128
NT = S // TCHUNK
TOK = B * TCHUNK


_i0 = np.int32(0)


def _routes(tok_u32):
    x = tok_u32
    x = x ^ (x >> 16)
    x = x * jnp.uint32(2246822507)
    x = x ^ (x >> 13)
    x = x * jnp.uint32(3266489909)
    x = x ^ (x >> 16)
    return (x & jnp.uint32(7)).astype(jnp.int32)


_NW = 32
_PER_W = (B * S) // _NW
_VREGS = _PER_W // 16


def _routes_sc(tok_flat_u32):
    mesh = plsc.VectorSubcoreMesh(core_axis_name="c", subcore_axis_name="s")

    @functools.partial(
        pl.kernel, mesh=mesh,
        out_type=jax.ShapeDtypeStruct((B * S,), jnp.int32),
        scratch_types=[
            pltpu.VMEM((_PER_W,), jnp.uint32),
            pltpu.VMEM((_PER_W,), jnp.int32),
        ],
    )
    def k(tok_hbm, out_hbm, tin, tout):
        wid = lax.axis_index("s") * 2 + lax.axis_index("c")
        base = wid * _PER_W
        pltpu.sync_copy(tok_hbm.at[pl.ds(base, _PER_W)], tin)
        for i in range(_VREGS):
            tout[pl.ds(i * 16, 16)] = _routes(tin[pl.ds(i * 16, 16)])
        pltpu.sync_copy(tout, out_hbm.at[pl.ds(base, _PER_W)])

    return k(tok_flat_u32)


def _dot_nt(a, b):
    return lax.dot_general(a, b, (((1,), (1,)), ((), ())),
                           preferred_element_type=jnp.float32)


def _shift_down(x, s, fill):
    rolled = pltpu.roll(x, jnp.int32(s), 0)
    tmod = lax.rem(lax.broadcasted_iota(jnp.int32, x.shape, 0),
                   jnp.int32(TCHUNK))
    return jnp.where(tmod >= s, rolled, fill)


def _ssm_body(tok_ref, x_ref, wi_ref, wsi_ref, wsot_ref, wot_ref, d_ref,
              out_ref, h_ref):
    t = pl.program_id(0)

    @pl.when(t == 0)
    def _():
        h_ref[...] = jnp.zeros_like(h_ref)

    r = tok_ref[0]
    xc = x_ref[...].reshape(TOK, DIM)

    u_all = _dot_nt(xc, wi_ref[...])
    sh = _dot_nt(xc, wsi_ref[...])
    sh = sh * jax.nn.sigmoid(sh)
    lane_sh = lax.broadcasted_iota(jnp.int32, (TOK, NE * SHID), 1) // SHID
    shm = jnp.where(lane_sh == r, sh, 0.0)
    sel = _dot_nt(shm, wsot_ref[...])

    lane_e = lax.broadcasted_iota(jnp.int32, (TOK, NE * SD), 1) // SD
    eq = lane_e == r

    u = jnp.zeros((TOK, SD), jnp.float32)
    for e in range(NE):
        u = u + jnp.where(r == e, u_all[:, e * SD:(e + 1) * SD], 0.0)

    a = jax.nn.sigmoid(sel[:, :SD])
    b = jnp.tanh(sel[:, SD:2 * SD])
    c = jnp.tanh(sel[:, 2 * SD:3 * SD])
    dd = jax.nn.sigmoid(sel[:, 3 * SD:])
    v = b * u

    a_rep = jnp.concatenate([a] * NE, axis=1)
    v_rep = jnp.concatenate([v] * NE, axis=1)
    A = jnp.where(eq, a_rep, 1.0)
    V = jnp.where(eq, v_rep, 0.0)

    s = 1
    while s < TCHUNK:
        Ash = _shift_down(A, s, 1.0)
        Vsh = _shift_down(V, s, 0.0)
        V = V + A * Vsh
        A = A * Ash
        s *= 2

    h_enter = jnp.broadcast_to(h_ref[...][:, None, :],
                               (B, TCHUNK, NE * SD)).reshape(TOK, NE * SD)
    h_all = V + A * h_enter
    h_ref[...] = h_all.reshape(B, TCHUNK, NE * SD)[:, TCHUNK - 1]

    h_sel = jnp.zeros((TOK, SD), jnp.float32)
    d_sel = jnp.zeros((TOK, SD), jnp.float32)
    for e in range(NE):
        m = r == e
        h_sel = h_sel + jnp.where(m, h_all[:, e * SD:(e + 1) * SD], 0.0)
        d_sel = d_sel + jnp.where(m, d_ref[e][None, :], 0.0)

    y = c * h_sel + d_sel * dd * u

    y_rep = jnp.concatenate([y] * NE, axis=1)
    y_wide = jnp.where(eq, y_rep, 0.0)
    out_acc = _dot_nt(y_wide, wot_ref[...])
    out_ref[...] = out_acc.reshape(B, TCHUNK, DIM)


def _to_f32(w):
    return w.astype(jnp.float32)
    if False:
        pass
    hi = (lax.bitcast_convert_type(w, jnp.uint64) >> 32).astype(jnp.uint32)
    s = hi & jnp.uint32(0x80000000)
    e11 = (hi >> 20) & jnp.uint32(0x7FF)
    m20 = hi & jnp.uint32(0xFFFFF)
    bits = s | ((e11 - jnp.uint32(896)) << 23) | (m20 << 3)
    return lax.bitcast_convert_type(jnp.where(e11 <= 896, s, bits),
                                    jnp.float32)


def _to_out_dtype(y, out_dtype):
    if out_dtype != jnp.float64:
        return y.astype(out_dtype)
    b = lax.bitcast_convert_type(y, jnp.uint32)
    s = b & jnp.uint32(0x80000000)
    e8 = (b >> 23) & jnp.uint32(0xFF)
    m = b & jnp.uint32(0x7FFFFF)
    hi = s | ((e8 + jnp.uint32(896)) << 20) | (m >> 3)
    lo = m << 29
    hi = jnp.where(e8 == 0, s, hi)
    lo = jnp.where(e8 == 0, jnp.uint32(0), lo)
    w64 = (hi.astype(jnp.uint64) << 32) | lo.astype(jnp.uint64)
    return lax.bitcast_convert_type(w64, jnp.float64)


def kernel(x, token_ids, W_in, W_si, W_so, W_out, d_param):
    out_dtype = jnp.result_type(W_out.dtype, x.dtype)
    routes = _routes_sc(token_ids.astype(jnp.uint32).reshape(B * S))
    tok_col = (routes.reshape(B, NT, TCHUNK).transpose(1, 0, 2)
               .reshape(NT, TOK, 1))
    x = _to_f32(x)
    W_in = _to_f32(W_in).reshape(NE * SD, DIM)
    W_si = _to_f32(W_si).reshape(NE * SHID, DIM)
    W_so = _to_f32(W_so).transpose(1, 0, 2).reshape(4 * SD, NE * SHID)
    W_out = _to_f32(W_out).transpose(1, 0, 2).reshape(DIM, NE * SD)
    d_param = _to_f32(d_param)
    out = pl.pallas_call(
        _ssm_body,
        grid=(NT,),
        in_specs=[
            pl.BlockSpec((1, TOK, 1), lambda t: (t, _i0, _i0)),
            pl.BlockSpec((B, TCHUNK, DIM), lambda t: (_i0, t, _i0)),
            pl.BlockSpec((NE * SD, DIM), lambda t: (_i0, _i0)),
            pl.BlockSpec((NE * SHID, DIM), lambda t: (_i0, _i0)),
            pl.BlockSpec((4 * SD, NE * SHID), lambda t: (_i0, _i0)),
            pl.BlockSpec((DIM, NE * SD), lambda t: (_i0, _i0)),
            pl.BlockSpec((NE, SD), lambda t: (_i0, _i0)),
        ],
        out_specs=pl.BlockSpec((B, TCHUNK, DIM), lambda t: (_i0, t, _i0)),
        out_shape=jax.ShapeDtypeStruct((B, S, DIM), jnp.float32),
        scratch_shapes=[pltpu.VMEM((B, NE * SD), jnp.float32)],
        compiler_params=pltpu.CompilerParams(
            dimension_semantics=("arbitrary",)),
    )(tok_col, x, W_in, W_si, W_so, W_out, d_param)
    return _to_out_dtype(out, out_dtype)


# speedup vs baseline: 1.2746x; 1.1071x over previous
import jax
import jax.numpy as jnp
import numpy as np
from jax import lax
from jax.experimental import pallas as pl
from jax.experimental.pallas import tpu as pltpu
from jax.experimental.pallas import tpu_sc as plsc
import functools

DIM = 1024
SD = 128
SHID = 256
NE = 8
B = 4
S = 2048
TCHUNK = 128
NT = S // TCHUNK
TOK = B * TCHUNK


_i0 = np.int32(0)


def _routes(tok_u32):
    x = tok_u32
    x = x ^ (x >> 16)
    x = x * jnp.uint32(2246822507)
    x = x ^ (x >> 13)
    x = x * jnp.uint32(3266489909)
    x = x ^ (x >> 16)
    return (x & jnp.uint32(7)).astype(jnp.int32)


_NW = 32
_PER_W = (B * S) // _NW
_VREGS = _PER_W // 16


def _routes_sc(tok_flat_u32):
    mesh = plsc.VectorSubcoreMesh(core_axis_name="c", subcore_axis_name="s")

    @functools.partial(
        pl.kernel, mesh=mesh,
        out_type=jax.ShapeDtypeStruct((B * S,), jnp.int32),
        scratch_types=[
            pltpu.VMEM((_PER_W,), jnp.uint32),
            pltpu.VMEM((_PER_W,), jnp.int32),
        ],
    )
    def k(tok_hbm, out_hbm, tin, tout):
        wid = lax.axis_index("s") * 2 + lax.axis_index("c")
        base = wid * _PER_W
        pltpu.sync_copy(tok_hbm.at[pl.ds(base, _PER_W)], tin)
        for i in range(_VREGS):
            tout[pl.ds(i * 16, 16)] = _routes(tin[pl.ds(i * 16, 16)])
        pltpu.sync_copy(tout, out_hbm.at[pl.ds(base, _PER_W)])

    return k(tok_flat_u32)


def _dot_nt(a, b):
    return lax.dot_general(a, b, (((1,), (1,)), ((), ())),
                           preferred_element_type=jnp.float32)


def _shift_down(x, s, fill):
    rolled = pltpu.roll(x, jnp.int32(s), 0)
    tmod = lax.rem(lax.broadcasted_iota(jnp.int32, x.shape, 0),
                   jnp.int32(TCHUNK))
    return jnp.where(tmod >= s, rolled, fill)


def _ssm_body(tok_ref, x_ref, wi_ref, wsi_ref, wsot_ref, wot_ref, d_ref,
              out_ref, h_ref):
    t = pl.program_id(0)

    @pl.when(t == 0)
    def _():
        h_ref[...] = jnp.zeros_like(h_ref)

    r = tok_ref[0]
    xc = x_ref[...].reshape(TOK, DIM)

    u_all = _dot_nt(xc, wi_ref[...])
    sh = _dot_nt(xc, wsi_ref[...])
    sh = sh * jax.nn.sigmoid(sh)
    lane_sh = lax.broadcasted_iota(jnp.int32, (TOK, NE * SHID), 1) // SHID
    shm = jnp.where(lane_sh == r, sh, 0.0)
    sel = _dot_nt(shm, wsot_ref[...])

    lane_e = lax.broadcasted_iota(jnp.int32, (TOK, NE * SD), 1) // SD
    eq = lane_e == r

    u = jnp.zeros((TOK, SD), jnp.float32)
    for e in range(NE):
        u = u + jnp.where(r == e, u_all[:, e * SD:(e + 1) * SD], 0.0)

    a = jax.nn.sigmoid(sel[:, :SD])
    b = jnp.tanh(sel[:, SD:2 * SD])
    c = jnp.tanh(sel[:, 2 * SD:3 * SD])
    dd = jax.nn.sigmoid(sel[:, 3 * SD:])
    v = b * u

    a_rep = jnp.concatenate([a] * NE, axis=1)
    v_rep = jnp.concatenate([v] * NE, axis=1)
    A = jnp.where(eq, a_rep, 1.0)
    V = jnp.where(eq, v_rep, 0.0)

    s = 1
    while s < TCHUNK:
        Ash = _shift_down(A, s, 1.0)
        Vsh = _shift_down(V, s, 0.0)
        V = V + A * Vsh
        A = A * Ash
        s *= 2

    h_enter = jnp.broadcast_to(h_ref[...][:, None, :],
                               (B, TCHUNK, NE * SD)).reshape(TOK, NE * SD)
    h_all = V + A * h_enter
    h_ref[...] = h_all.reshape(B, TCHUNK, NE * SD)[:, TCHUNK - 1]

    h_sel = jnp.zeros((TOK, SD), jnp.float32)
    d_sel = jnp.zeros((TOK, SD), jnp.float32)
    for e in range(NE):
        m = r == e
        h_sel = h_sel + jnp.where(m, h_all[:, e * SD:(e + 1) * SD], 0.0)
        d_sel = d_sel + jnp.where(m, d_ref[e][None, :], 0.0)

    y = c * h_sel + d_sel * dd * u

    y_rep = jnp.concatenate([y] * NE, axis=1)
    y_wide = jnp.where(eq, y_rep, 0.0)
    out_acc = _dot_nt(y_wide, wot_ref[...])
    out_ref[...] = out_acc.reshape(B, TCHUNK, DIM)


def _to_f32(w):
    return w.astype(jnp.float32)


def _to_out_dtype(y, out_dtype):
    return y.astype(out_dtype)


def kernel(x, token_ids, W_in, W_si, W_so, W_out, d_param):
    out_dtype = jnp.result_type(W_out.dtype, x.dtype)
    routes = _routes_sc(token_ids.astype(jnp.uint32).reshape(B * S))
    tok_col = (routes.reshape(B, NT, TCHUNK).transpose(1, 0, 2)
               .reshape(NT, TOK, 1))
    x = _to_f32(x)
    W_in = _to_f32(W_in).reshape(NE * SD, DIM)
    W_si = _to_f32(W_si).reshape(NE * SHID, DIM)
    W_so = _to_f32(W_so).transpose(1, 0, 2).reshape(4 * SD, NE * SHID)
    W_out = _to_f32(W_out).transpose(1, 0, 2).reshape(DIM, NE * SD)
    d_param = _to_f32(d_param)
    out = pl.pallas_call(
        _ssm_body,
        grid=(NT,),
        in_specs=[
            pl.BlockSpec((1, TOK, 1), lambda t: (t, _i0, _i0)),
            pl.BlockSpec((B, TCHUNK, DIM), lambda t: (_i0, t, _i0)),
            pl.BlockSpec((NE * SD, DIM), lambda t: (_i0, _i0)),
            pl.BlockSpec((NE * SHID, DIM), lambda t: (_i0, _i0)),
            pl.BlockSpec((4 * SD, NE * SHID), lambda t: (_i0, _i0)),
            pl.BlockSpec((DIM, NE * SD), lambda t: (_i0, _i0)),
            pl.BlockSpec((NE, SD), lambda t: (_i0, _i0)),
        ],
        out_specs=pl.BlockSpec((B, TCHUNK, DIM), lambda t: (_i0, t, _i0)),
        out_shape=jax.ShapeDtypeStruct((B, S, DIM), jnp.float32),
        scratch_shapes=[pltpu.VMEM((B, NE * SD), jnp.float32)],
        compiler_params=pltpu.CompilerParams(
            dimension_semantics=("arbitrary",)),
    )(tok_col, x, W_in, W_si, W_so, W_out, d_param)
    return _to_out_dtype(out, out_dtype)
